# baseline (speedup 1.0000x reference)
import dataclasses
import functools

import jax
import jax.numpy as jnp
from jax import lax
from jax.experimental import pallas as pl
from jax.experimental.pallas import tpu as pltpu
from jax.experimental.pallas import tpu_sc as plsc

_UNROLLED_ITERS = 7
_MAX_EXTRA_ITERS = 24


def _sparsemax_block(x_ref, o_ref):
    x = x_ref[...]

    def feval(t):
        return jnp.sum(jnp.maximum(x - t, 0.0), axis=-1, keepdims=True) - 1.0

    def secant(t0, f0, t1, f1):
        denom = f1 - f0
        return jnp.where(denom < 0.0, t1 - f1 * (t1 - t0) / denom, t1)

    m = jnp.max(x, axis=-1, keepdims=True)
    t0 = m - 2.0
    f0 = feval(t0)
    t1 = m - 1.0
    f1 = feval(t1)
    for _ in range(_UNROLLED_ITERS):
        t2 = secant(t0, f0, t1, f1)
        t0, f0, t1, f1 = t1, f1, t2, feval(t2)

    def cond(carry):
        k = carry[0]
        changed = carry[5]
        return jnp.logical_and(k < _MAX_EXTRA_ITERS, changed)

    def body(carry):
        k, t0, f0, t1, f1, _ = carry
        t2 = secant(t0, f0, t1, f1)
        return k + 1, t1, f1, t2, feval(t2), jnp.any(t2 != t1)

    _, _, _, t1, _, _ = jax.lax.while_loop(
        cond, body, (0, t0, f0, t1, f1, jnp.bool_(True))
    )
    o_ref[...] = jnp.maximum(x - t1, 0.0)


_ROW_BLOCK = 8


_SC_LANES = 16
_SC_CORES = 2
_SC_SUBCORES = 16
_SC_WORKERS = _SC_CORES * _SC_SUBCORES


def _sc_sdiv(a, b):
    q = jnp.full((_SC_LANES,), a) / jnp.full((_SC_LANES,), b)
    return jnp.max(q)


_SC_UNROLL = 4


def _sc_row_sparsemax(row_v, out_v, smax_v, idx_ref, sup_ref, n_chunks):
    L = _SC_LANES
    n_sup = n_chunks // L
    zeros = jnp.zeros((L,), jnp.float32)
    ninf = jnp.full((L,), -jnp.inf, jnp.float32)

    def sup_body(j, gmax):
        base = j * (L * L)
        accs = [row_v[pl.ds(base + u * L, L)] for u in range(4)]
        for u in range(4, L, 4):
            for a in range(4):
                accs[a] = jnp.maximum(
                    accs[a], row_v[pl.ds(base + (u + a) * L, L)]
                )
        sm = jnp.maximum(
            jnp.maximum(accs[0], accs[1]), jnp.maximum(accs[2], accs[3])
        )
        smax_v[pl.ds(j * L, L)] = sm
        return jnp.maximum(gmax, sm)

    gmax = lax.fori_loop(0, n_sup, sup_body, ninf)
    tau0 = jnp.max(gmax) - 1.0

    def sup_screen(j, nsup):
        anyc = jnp.any(smax_v[pl.ds(j * L, L)] > tau0)
        sup_ref[nsup] = j
        return nsup + jnp.where(anyc, 1, 0)

    nsup = lax.fori_loop(0, n_sup, sup_screen, 0)

    def chunk_screen(jj, nch):
        j = sup_ref[jj]

        def inner(u, nch):
            ci = j * L + u
            anyc = jnp.any(row_v[pl.ds(ci * L, L)] > tau0)
            idx_ref[nch] = ci
            return nch + jnp.where(anyc, 1, 0)

        return lax.fori_loop(0, L, inner, nch)

    nch = lax.fori_loop(0, nsup, chunk_screen, 0)

    def newton(tau):
        def nb(j, carry):
            s, c = carry
            v = row_v[pl.ds(idx_ref[j] * L, L)]
            mask = v > tau
            s = s + jnp.where(mask, v - tau, 0.0)
            c = c + jnp.where(mask, 1.0, 0.0)
            return s, c

        s16, c16 = lax.fori_loop(0, nch, nb, (zeros, zeros))
        return tau + _sc_sdiv(jnp.sum(s16) - 1.0, jnp.sum(c16))

    def w_cond(carry):
        k, _, changed = carry
        return jnp.logical_and(k < 40, changed)

    def w_body(carry):
        k, tau, _ = carry
        t2 = newton(tau)
        return k + 1, t2, t2 != tau

    _, tau, _ = lax.while_loop(w_cond, w_body, (0, tau0, jnp.bool_(True)))

    def out_body(j, carry):
        sl = pl.ds(idx_ref[j] * L, L)
        out_v[sl] = jnp.maximum(row_v[sl] - tau, 0.0)
        return carry

    lax.fori_loop(0, nch, out_body, 0)
    return nch


def _sc_sparsemax(x):
    rows, cols = x.shape
    n_chunks = cols // _SC_LANES
    rpw = -(-rows // _SC_WORKERS)
    mesh = plsc.VectorSubcoreMesh(core_axis_name="c", subcore_axis_name="s")
    cp = pltpu.CompilerParams()
    if "needs_layout_passes" in pltpu.CompilerParams.__dataclass_fields__:
        cp = dataclasses.replace(cp, needs_layout_passes=False)

    @functools.partial(
        pl.kernel,
        out_type=jax.ShapeDtypeStruct((rows, cols), x.dtype),
        mesh=mesh,
        compiler_params=cp,
        scratch_types=[
            pltpu.VMEM((cols,), jnp.float32),
            pltpu.VMEM((cols,), jnp.float32),
            pltpu.VMEM((n_chunks,), jnp.float32),
            pltpu.SMEM((n_chunks,), jnp.int32),
            pltpu.SMEM((max(n_chunks // _SC_LANES, 8),), jnp.int32),
        ],
    )
    def k(x_hbm, o_hbm, row_v, out_v, smax_v, idx_ref, sup_ref):
        wid = lax.axis_index("s") * _SC_CORES + lax.axis_index("c")
        L = _SC_LANES
        zeros = jnp.zeros((L,), jnp.float32)

        def zero_body(i, carry):
            out_v[pl.ds(i * L, L)] = zeros
            return carry

        lax.fori_loop(0, n_chunks, zero_body, 0)

        def do_row(row):
            pltpu.sync_copy(x_hbm.at[row], row_v)
            nch = _sc_row_sparsemax(
                row_v, out_v, smax_v, idx_ref, sup_ref, n_chunks
            )
            pltpu.sync_copy(out_v, o_hbm.at[row])

            def rezero_body(j, carry):
                out_v[pl.ds(idx_ref[j] * L, L)] = zeros
                return carry

            lax.fori_loop(0, nch, rezero_body, 0)

        for r in range(rpw):
            row = r * _SC_WORKERS + wid
            if rows % _SC_WORKERS == 0:
                do_row(row)
            else:
                pl.when(row < rows)(lambda row=row: do_row(row))

    return k(x)


_SC_ROWS = 32


def _hybrid_kernel(x):
    rows = x.shape[0]
    n_tc = rows - _SC_ROWS
    tc_out = _tc_kernel(lax.slice_in_dim(x, 0, n_tc, axis=0))
    sc_out = _sc_sparsemax(lax.slice_in_dim(x, n_tc, rows, axis=0))
    return jnp.concatenate([tc_out, sc_out], axis=0)


@functools.partial(jax.jit, static_argnames=())
def kernel(x):
    return _hybrid_kernel(x)


def _tc_kernel(x):
    return pl.pallas_call(
        _sparsemax_block,
        out_shape=jax.ShapeDtypeStruct(x.shape, x.dtype),
    )(x)


# speedup vs baseline: 1.0382x; 1.0382x over previous
import dataclasses
import functools

import jax
import jax.numpy as jnp
from jax import lax
from jax.experimental import pallas as pl
from jax.experimental.pallas import tpu as pltpu
from jax.experimental.pallas import tpu_sc as plsc

_UNROLLED_ITERS = 7
_MAX_EXTRA_ITERS = 24


def _sparsemax_block(x_ref, o_ref):
    x = x_ref[...]

    def feval(t):
        return jnp.sum(jnp.maximum(x - t, 0.0), axis=-1, keepdims=True) - 1.0

    def secant(t0, f0, t1, f1):
        denom = f1 - f0
        return jnp.where(denom < 0.0, t1 - f1 * (t1 - t0) / denom, t1)

    m = jnp.max(x, axis=-1, keepdims=True)
    t0 = m - 2.0
    f0 = feval(t0)
    t1 = m - 1.0
    f1 = feval(t1)
    for _ in range(_UNROLLED_ITERS):
        t2 = secant(t0, f0, t1, f1)
        t0, f0, t1, f1 = t1, f1, t2, feval(t2)

    def cond(carry):
        k = carry[0]
        changed = carry[5]
        return jnp.logical_and(k < _MAX_EXTRA_ITERS, changed)

    def body(carry):
        k, t0, f0, t1, f1, _ = carry
        t2 = secant(t0, f0, t1, f1)
        return k + 1, t1, f1, t2, feval(t2), jnp.any(t2 != t1)

    _, _, _, t1, _, _ = jax.lax.while_loop(
        cond, body, (0, t0, f0, t1, f1, jnp.bool_(True))
    )
    o_ref[...] = jnp.maximum(x - t1, 0.0)


_ROW_BLOCK = 8


_SC_LANES = 16
_SC_CORES = 2
_SC_SUBCORES = 16
_SC_WORKERS = _SC_CORES * _SC_SUBCORES


def _sc_sdiv(a, b):
    q = jnp.full((_SC_LANES,), a) / jnp.full((_SC_LANES,), b)
    return jnp.max(q)


_SC_UNROLL = 4


def _sc_row_sparsemax(row_v, out_v, smax_v, idx_ref, sup_ref, n_chunks):
    L = _SC_LANES
    n_sup = n_chunks // L
    zeros = jnp.zeros((L,), jnp.float32)
    ninf = jnp.full((L,), -jnp.inf, jnp.float32)

    def sup_body(j, gmax):
        base = j * (L * L)
        accs = [row_v[pl.ds(base + u * L, L)] for u in range(4)]
        for u in range(4, L, 4):
            for a in range(4):
                accs[a] = jnp.maximum(
                    accs[a], row_v[pl.ds(base + (u + a) * L, L)]
                )
        sm = jnp.maximum(
            jnp.maximum(accs[0], accs[1]), jnp.maximum(accs[2], accs[3])
        )
        smax_v[pl.ds(j * L, L)] = sm
        return jnp.maximum(gmax, sm)

    gmax = lax.fori_loop(0, n_sup, sup_body, ninf)
    tau0 = jnp.max(gmax) - 1.0

    def sup_screen(j, nsup):
        anyc = jnp.any(smax_v[pl.ds(j * L, L)] > tau0)
        sup_ref[nsup] = j
        return nsup + jnp.where(anyc, 1, 0)

    nsup = lax.fori_loop(0, n_sup, sup_screen, 0)

    def chunk_screen(jj, nch):
        j = sup_ref[jj]

        def inner(u, nch):
            ci = j * L + u
            anyc = jnp.any(row_v[pl.ds(ci * L, L)] > tau0)
            idx_ref[nch] = ci
            return nch + jnp.where(anyc, 1, 0)

        return lax.fori_loop(0, L, inner, nch)

    nch = lax.fori_loop(0, nsup, chunk_screen, 0)

    def newton(tau):
        def nb(j, carry):
            s, c = carry
            v = row_v[pl.ds(idx_ref[j] * L, L)]
            mask = v > tau
            s = s + jnp.where(mask, v - tau, 0.0)
            c = c + jnp.where(mask, 1.0, 0.0)
            return s, c

        s16, c16 = lax.fori_loop(0, nch, nb, (zeros, zeros))
        return tau + _sc_sdiv(jnp.sum(s16) - 1.0, jnp.sum(c16))

    def w_cond(carry):
        k, _, changed = carry
        return jnp.logical_and(k < 40, changed)

    def w_body(carry):
        k, tau, _ = carry
        t2 = newton(tau)
        return k + 1, t2, t2 != tau

    _, tau, _ = lax.while_loop(w_cond, w_body, (0, tau0, jnp.bool_(True)))

    def out_body(j, carry):
        sl = pl.ds(idx_ref[j] * L, L)
        out_v[sl] = jnp.maximum(row_v[sl] - tau, 0.0)
        return carry

    lax.fori_loop(0, nch, out_body, 0)
    return nch


def _sc_sparsemax(x):
    rows, cols = x.shape
    n_chunks = cols // _SC_LANES
    rpw = -(-rows // _SC_WORKERS)
    mesh = plsc.VectorSubcoreMesh(core_axis_name="c", subcore_axis_name="s")
    cp = pltpu.CompilerParams()
    if "needs_layout_passes" in pltpu.CompilerParams.__dataclass_fields__:
        cp = dataclasses.replace(cp, needs_layout_passes=False)

    @functools.partial(
        pl.kernel,
        out_type=jax.ShapeDtypeStruct((rows, cols), x.dtype),
        mesh=mesh,
        compiler_params=cp,
        scratch_types=[
            pltpu.VMEM((cols,), jnp.float32),
            pltpu.VMEM((cols,), jnp.float32),
            pltpu.VMEM((n_chunks,), jnp.float32),
            pltpu.SMEM((n_chunks,), jnp.int32),
            pltpu.SMEM((max(n_chunks // _SC_LANES, 8),), jnp.int32),
            pltpu.SemaphoreType.DMA,
        ],
    )
    def k(x_hbm, o_hbm, row_v, out_v, smax_v, idx_ref, sup_ref, sem):
        wid = lax.axis_index("s") * _SC_CORES + lax.axis_index("c")
        L = _SC_LANES
        zeros = jnp.zeros((L,), jnp.float32)

        def do_row(row, first, last):
            cp = pltpu.async_copy(x_hbm.at[row], row_v, sem)
            if first:

                def zero_body(i, carry):
                    out_v[pl.ds(i * L, L)] = zeros
                    return carry

                lax.fori_loop(0, n_chunks, zero_body, 0)
            cp.wait()
            nch = _sc_row_sparsemax(
                row_v, out_v, smax_v, idx_ref, sup_ref, n_chunks
            )
            pltpu.sync_copy(out_v, o_hbm.at[row])
            if not last:

                def rezero_body(j, carry):
                    out_v[pl.ds(idx_ref[j] * L, L)] = zeros
                    return carry

                lax.fori_loop(0, nch, rezero_body, 0)

        for r in range(rpw):
            row = r * _SC_WORKERS + wid
            first, last = r == 0, r == rpw - 1
            if rows % _SC_WORKERS == 0:
                do_row(row, first, last)
            else:
                pl.when(row < rows)(
                    lambda row=row, f=first, l=last: do_row(row, f, l)
                )

    return k(x)


_SC_ROWS = 32


def _hybrid_kernel(x):
    rows = x.shape[0]
    n_tc = rows - _SC_ROWS
    tc_out = _tc_kernel(lax.slice_in_dim(x, 0, n_tc, axis=0))
    sc_out = _sc_sparsemax(lax.slice_in_dim(x, n_tc, rows, axis=0))
    return jnp.concatenate([tc_out, sc_out], axis=0)


@functools.partial(jax.jit, static_argnames=())
def kernel(x):
    return _hybrid_kernel(x)


def _tc_kernel(x):
    return pl.pallas_call(
        _sparsemax_block,
        out_shape=jax.ShapeDtypeStruct(x.shape, x.dtype),
    )(x)


# speedup vs baseline: 1.0870x; 1.0471x over previous
import dataclasses
import functools

import jax
import jax.numpy as jnp
from jax import lax
from jax.experimental import pallas as pl
from jax.experimental.pallas import tpu as pltpu
from jax.experimental.pallas import tpu_sc as plsc

_UNROLLED_ITERS = 7
_MAX_EXTRA_ITERS = 24


def _sparsemax_block(x_ref, o_ref):
    x = x_ref[...]

    def feval(t):
        return jnp.sum(jnp.maximum(x - t, 0.0), axis=-1, keepdims=True) - 1.0

    def secant(t0, f0, t1, f1):
        denom = f1 - f0
        return jnp.where(denom < 0.0, t1 - f1 * (t1 - t0) / denom, t1)

    m = jnp.max(x, axis=-1, keepdims=True)
    t0 = m - 2.0
    f0 = feval(t0)
    t1 = m - 1.0
    f1 = feval(t1)
    for _ in range(_UNROLLED_ITERS):
        t2 = secant(t0, f0, t1, f1)
        t0, f0, t1, f1 = t1, f1, t2, feval(t2)

    def cond(carry):
        k = carry[0]
        changed = carry[5]
        return jnp.logical_and(k < _MAX_EXTRA_ITERS, changed)

    def body(carry):
        k, t0, f0, t1, f1, _ = carry
        t2 = secant(t0, f0, t1, f1)
        return k + 1, t1, f1, t2, feval(t2), jnp.any(t2 != t1)

    _, _, _, t1, _, _ = jax.lax.while_loop(
        cond, body, (0, t0, f0, t1, f1, jnp.bool_(True))
    )
    o_ref[...] = jnp.maximum(x - t1, 0.0)


_ROW_BLOCK = 8


_SC_LANES = 16
_SC_CORES = 2
_SC_SUBCORES = 16
_SC_WORKERS = _SC_CORES * _SC_SUBCORES


def _sc_sdiv(a, b):
    q = jnp.full((_SC_LANES,), a) / jnp.full((_SC_LANES,), b)
    return jnp.max(q)


_SC_UNROLL = 4


def _sc_row_sparsemax(row_v, out_v, smax_v, idx_ref, sup_ref, n_chunks):
    L = _SC_LANES
    n_sup = n_chunks // L
    zeros = jnp.zeros((L,), jnp.float32)
    ninf = jnp.full((L,), -jnp.inf, jnp.float32)

    def sup_body(j, gmax):
        base = j * (L * L)
        accs = [row_v[pl.ds(base + u * L, L)] for u in range(4)]
        for u in range(4, L, 4):
            for a in range(4):
                accs[a] = jnp.maximum(
                    accs[a], row_v[pl.ds(base + (u + a) * L, L)]
                )
        sm = jnp.maximum(
            jnp.maximum(accs[0], accs[1]), jnp.maximum(accs[2], accs[3])
        )
        smax_v[pl.ds(j * L, L)] = sm
        return jnp.maximum(gmax, sm)

    gmax = lax.fori_loop(0, n_sup, sup_body, ninf)
    tau0 = jnp.max(gmax) - 1.0

    def sup_screen(j, nsup):
        anyc = jnp.any(smax_v[pl.ds(j * L, L)] > tau0)
        sup_ref[nsup] = j
        return nsup + jnp.where(anyc, 1, 0)

    nsup = lax.fori_loop(0, n_sup, sup_screen, 0)

    def chunk_screen(jj, nch):
        j = sup_ref[jj]

        def inner(u, nch):
            ci = j * L + u
            anyc = jnp.any(row_v[pl.ds(ci * L, L)] > tau0)
            idx_ref[nch] = ci
            return nch + jnp.where(anyc, 1, 0)

        return lax.fori_loop(0, L, inner, nch)

    nch = lax.fori_loop(0, nsup, chunk_screen, 0)

    def newton(tau):
        def nb(j, carry):
            s, c = carry
            v = row_v[pl.ds(idx_ref[j] * L, L)]
            mask = v > tau
            s = s + jnp.where(mask, v - tau, 0.0)
            c = c + jnp.where(mask, 1.0, 0.0)
            return s, c

        s16, c16 = lax.fori_loop(0, nch, nb, (zeros, zeros))
        return tau + _sc_sdiv(jnp.sum(s16) - 1.0, jnp.sum(c16))

    def w_cond(carry):
        k, _, changed = carry
        return jnp.logical_and(k < 40, changed)

    def w_body(carry):
        k, tau, _ = carry
        t2 = newton(tau)
        return k + 1, t2, t2 != tau

    _, tau, _ = lax.while_loop(w_cond, w_body, (0, tau0, jnp.bool_(True)))

    def out_body(j, carry):
        sl = pl.ds(idx_ref[j] * L, L)
        out_v[sl] = jnp.maximum(row_v[sl] - tau, 0.0)
        return carry

    lax.fori_loop(0, nch, out_body, 0)
    return nch


def _sc_sparsemax(x, row_start=0, n_rows=None):
    rows, cols = x.shape
    if n_rows is None:
        n_rows = rows
    n_chunks = cols // _SC_LANES
    rpw = -(-n_rows // _SC_WORKERS)
    mesh = plsc.VectorSubcoreMesh(core_axis_name="c", subcore_axis_name="s")
    cp = pltpu.CompilerParams()
    if "needs_layout_passes" in pltpu.CompilerParams.__dataclass_fields__:
        cp = dataclasses.replace(cp, needs_layout_passes=False)

    @functools.partial(
        pl.kernel,
        out_type=jax.ShapeDtypeStruct((n_rows, cols), x.dtype),
        mesh=mesh,
        compiler_params=cp,
        scratch_types=[
            pltpu.VMEM((cols,), jnp.float32),
            pltpu.VMEM((cols,), jnp.float32),
            pltpu.VMEM((n_chunks,), jnp.float32),
            pltpu.SMEM((n_chunks,), jnp.int32),
            pltpu.SMEM((max(n_chunks // _SC_LANES, 8),), jnp.int32),
            pltpu.SemaphoreType.DMA,
        ],
    )
    def k(x_hbm, o_hbm, row_v, out_v, smax_v, idx_ref, sup_ref, sem):
        wid = lax.axis_index("s") * _SC_CORES + lax.axis_index("c")
        L = _SC_LANES
        zeros = jnp.zeros((L,), jnp.float32)

        def do_row(row, first, last):
            cp = pltpu.async_copy(x_hbm.at[row_start + row], row_v, sem)
            if first:

                def zero_body(i, carry):
                    base = i * (16 * L)
                    for u in range(16):
                        out_v[pl.ds(base + u * L, L)] = zeros
                    return carry

                lax.fori_loop(0, n_chunks // 16, zero_body, 0)
            cp.wait()
            nch = _sc_row_sparsemax(
                row_v, out_v, smax_v, idx_ref, sup_ref, n_chunks
            )
            pltpu.sync_copy(out_v, o_hbm.at[row])
            if not last:

                def rezero_body(j, carry):
                    out_v[pl.ds(idx_ref[j] * L, L)] = zeros
                    return carry

                lax.fori_loop(0, nch, rezero_body, 0)

        for r in range(rpw):
            row = r * _SC_WORKERS + wid
            first, last = r == 0, r == rpw - 1
            if n_rows % _SC_WORKERS == 0:
                do_row(row, first, last)
            else:
                pl.when(row < n_rows)(
                    lambda row=row, f=first, l=last: do_row(row, f, l)
                )

    return k(x)


_SC_ROWS = 32


def _hybrid_kernel(x):
    rows, cols = x.shape
    n_tc = rows - _SC_ROWS
    tc_out = pl.pallas_call(
        _sparsemax_block,
        grid=(1,),
        in_specs=[pl.BlockSpec((n_tc, cols), lambda i: (0, 0))],
        out_specs=pl.BlockSpec((n_tc, cols), lambda i: (0, 0)),
        out_shape=jax.ShapeDtypeStruct((n_tc, cols), x.dtype),
    )(x)
    sc_out = _sc_sparsemax(x, row_start=n_tc, n_rows=_SC_ROWS)
    return jnp.concatenate([tc_out, sc_out], axis=0)


@functools.partial(jax.jit, static_argnames=())
def kernel(x):
    return _hybrid_kernel(x)


def _tc_kernel(x):
    return pl.pallas_call(
        _sparsemax_block,
        out_shape=jax.ShapeDtypeStruct(x.shape, x.dtype),
    )(x)


# speedup vs baseline: 1.1144x; 1.0251x over previous
import dataclasses
import functools

import jax
import jax.numpy as jnp
from jax import lax
from jax.experimental import pallas as pl
from jax.experimental.pallas import tpu as pltpu
from jax.experimental.pallas import tpu_sc as plsc

_UNROLLED_ITERS = 7
_MAX_EXTRA_ITERS = 24


def _sparsemax_block(x_ref, o_ref):
    x = x_ref[...]

    def feval(t):
        return jnp.sum(jnp.maximum(x - t, 0.0), axis=-1, keepdims=True) - 1.0

    def secant(t0, f0, t1, f1):
        denom = f1 - f0
        return jnp.where(denom < 0.0, t1 - f1 * (t1 - t0) / denom, t1)

    m = jnp.max(x, axis=-1, keepdims=True)
    t0 = m - 2.0
    f0 = feval(t0)
    t1 = m - 1.0
    f1 = feval(t1)
    for _ in range(_UNROLLED_ITERS):
        t2 = secant(t0, f0, t1, f1)
        t0, f0, t1, f1 = t1, f1, t2, feval(t2)

    def cond(carry):
        k = carry[0]
        changed = carry[5]
        return jnp.logical_and(k < _MAX_EXTRA_ITERS, changed)

    def body(carry):
        k, t0, f0, t1, f1, _ = carry
        t2 = secant(t0, f0, t1, f1)
        return k + 1, t1, f1, t2, feval(t2), jnp.any(t2 != t1)

    _, _, _, t1, _, _ = jax.lax.while_loop(
        cond, body, (0, t0, f0, t1, f1, jnp.bool_(True))
    )
    o_ref[...] = jnp.maximum(x - t1, 0.0)


_ROW_BLOCK = 8


_SC_LANES = 16
_SC_CORES = 2
_SC_SUBCORES = 16
_SC_WORKERS = _SC_CORES * _SC_SUBCORES


def _sc_sdiv(a, b):
    q = jnp.full((_SC_LANES,), a) / jnp.full((_SC_LANES,), b)
    return jnp.max(q)


_SC_UNROLL = 4


def _sc_row_sparsemax(row_v, out_v, smax_v, idx_ref, sup_ref, n_chunks):
    L = _SC_LANES
    n_sup = n_chunks // L
    zeros = jnp.zeros((L,), jnp.float32)
    ninf = jnp.full((L,), -jnp.inf, jnp.float32)

    def sup_body(j, gmax):
        base = j * (L * L)
        accs = [row_v[pl.ds(base + u * L, L)] for u in range(4)]
        for u in range(4, L, 4):
            for a in range(4):
                accs[a] = jnp.maximum(
                    accs[a], row_v[pl.ds(base + (u + a) * L, L)]
                )
        sm = jnp.maximum(
            jnp.maximum(accs[0], accs[1]), jnp.maximum(accs[2], accs[3])
        )
        smax_v[pl.ds(j * L, L)] = sm
        return jnp.maximum(gmax, sm)

    gmax = lax.fori_loop(0, n_sup, sup_body, ninf)
    tau0 = jnp.max(gmax) - 1.0

    nsup = 0
    for j in range(n_sup):
        anyc = jnp.any(smax_v[pl.ds(j * L, L)] > tau0)
        sup_ref[nsup] = j
        nsup = nsup + jnp.where(anyc, 1, 0)

    def chunk_screen(jj, nch):
        j = sup_ref[jj]

        def inner(u, nch):
            ci = j * L + u
            anyc = jnp.any(row_v[pl.ds(ci * L, L)] > tau0)
            idx_ref[nch] = ci
            return nch + jnp.where(anyc, 1, 0)

        return lax.fori_loop(0, L, inner, nch)

    nch = lax.fori_loop(0, nsup, chunk_screen, 0)

    def newton(tau):
        def nb(j, carry):
            s, c = carry
            v = row_v[pl.ds(idx_ref[j] * L, L)]
            mask = v > tau
            s = s + jnp.where(mask, v - tau, 0.0)
            c = c + jnp.where(mask, 1.0, 0.0)
            return s, c

        s16, c16 = lax.fori_loop(0, nch, nb, (zeros, zeros))
        return tau + _sc_sdiv(jnp.sum(s16) - 1.0, jnp.sum(c16))

    def w_cond(carry):
        k, _, changed = carry
        return jnp.logical_and(k < 40, changed)

    def w_body(carry):
        k, tau, _ = carry
        t2 = newton(tau)
        return k + 1, t2, t2 != tau

    _, tau, _ = lax.while_loop(w_cond, w_body, (0, tau0, jnp.bool_(True)))

    def out_body(j, carry):
        sl = pl.ds(idx_ref[j] * L, L)
        out_v[sl] = jnp.maximum(row_v[sl] - tau, 0.0)
        return carry

    lax.fori_loop(0, nch, out_body, 0)
    return nch


def _sc_sparsemax(x, row_start=0, n_rows=None):
    rows, cols = x.shape
    if n_rows is None:
        n_rows = rows
    n_chunks = cols // _SC_LANES
    rpw = -(-n_rows // _SC_WORKERS)
    mesh = plsc.VectorSubcoreMesh(core_axis_name="c", subcore_axis_name="s")
    cp = pltpu.CompilerParams()
    if "needs_layout_passes" in pltpu.CompilerParams.__dataclass_fields__:
        cp = dataclasses.replace(cp, needs_layout_passes=False)

    @functools.partial(
        pl.kernel,
        out_type=jax.ShapeDtypeStruct((rows, cols), x.dtype),
        mesh=mesh,
        compiler_params=cp,
        scratch_types=[
            pltpu.VMEM((cols,), jnp.float32),
            pltpu.VMEM((cols,), jnp.float32),
            pltpu.VMEM((n_chunks,), jnp.float32),
            pltpu.SMEM((n_chunks,), jnp.int32),
            pltpu.SMEM((max(n_chunks // _SC_LANES, 8),), jnp.int32),
            pltpu.SemaphoreType.DMA,
        ],
    )
    def k(x_hbm, o_hbm, row_v, out_v, smax_v, idx_ref, sup_ref, sem):
        wid = lax.axis_index("s") * _SC_CORES + lax.axis_index("c")
        L = _SC_LANES
        zeros = jnp.zeros((L,), jnp.float32)

        def do_row(row, first, last):
            cp = pltpu.async_copy(x_hbm.at[row_start + row], row_v, sem)
            if first:

                def zero_body(i, carry):
                    base = i * (16 * L)
                    for u in range(16):
                        out_v[pl.ds(base + u * L, L)] = zeros
                    return carry

                lax.fori_loop(0, n_chunks // 16, zero_body, 0)
            cp.wait()
            nch = _sc_row_sparsemax(
                row_v, out_v, smax_v, idx_ref, sup_ref, n_chunks
            )
            pltpu.sync_copy(out_v, o_hbm.at[row_start + row])
            if not last:

                def rezero_body(j, carry):
                    out_v[pl.ds(idx_ref[j] * L, L)] = zeros
                    return carry

                lax.fori_loop(0, nch, rezero_body, 0)

        for r in range(rpw):
            row = r * _SC_WORKERS + wid
            first, last = r == 0, r == rpw - 1
            if n_rows % _SC_WORKERS == 0:
                do_row(row, first, last)
            else:
                pl.when(row < n_rows)(
                    lambda row=row, f=first, l=last: do_row(row, f, l)
                )

    return k(x)


_SC_ROWS = 32


def _hybrid_kernel(x):
    rows, cols = x.shape
    n_tc = rows - _SC_ROWS
    tc_out = pl.pallas_call(
        _sparsemax_block,
        grid=(1,),
        in_specs=[pl.BlockSpec((n_tc, cols), lambda i: (0, 0))],
        out_specs=pl.BlockSpec((n_tc, cols), lambda i: (0, 0)),
        out_shape=jax.ShapeDtypeStruct((n_tc, cols), x.dtype),
    )(x)
    sc_out = _sc_sparsemax(x, row_start=n_tc, n_rows=_SC_ROWS)

    def merge_body(t_ref, _, o_ref):
        o_ref[...] = t_ref[...]

    return pl.pallas_call(
        merge_body,
        grid=(1,),
        in_specs=[
            pl.BlockSpec((n_tc, cols), lambda i: (0, 0)),
            pl.BlockSpec(memory_space=pl.ANY),
        ],
        out_specs=pl.BlockSpec((n_tc, cols), lambda i: (0, 0)),
        out_shape=jax.ShapeDtypeStruct((rows, cols), x.dtype),
        input_output_aliases={1: 0},
    )(tc_out, sc_out)


@functools.partial(jax.jit, static_argnames=())
def kernel(x):
    return _hybrid_kernel(x)


def _tc_kernel(x):
    return pl.pallas_call(
        _sparsemax_block,
        out_shape=jax.ShapeDtypeStruct(x.shape, x.dtype),
    )(x)
